# baseline (speedup 1.0000x reference)
import functools

import jax
import jax.numpy as jnp
from jax import lax
from jax.experimental import pallas as pl
from jax.experimental.pallas import tpu as pltpu
from jax.experimental.pallas import tpu_sc as plsc

N_NODES = 10000
N_EDGES = 320000
D_FEAT = 128

NW = 32
E_PER_W = N_EDGES // NW
C = 80
NCHUNK_W = E_PER_W // C
NBUF = 5
NGROUP = NCHUNK_W // NBUF

_mesh = plsc.VectorSubcoreMesh(core_axis_name="c", subcore_axis_name="s")


def _neg_body(x_ref, o_ref):
    o_ref[...] = -x_ref[...]


def _negate(x):
    return pl.pallas_call(
        _neg_body,
        out_shape=jax.ShapeDtypeStruct((N_NODES, D_FEAT), jnp.float32),
    )(x)


_scratch = [
    pltpu.VMEM((NCHUNK_W, C), jnp.int32),
    pltpu.VMEM((NCHUNK_W, C), jnp.int32),
]
for _b in range(NBUF):
    _scratch += [
        pltpu.VMEM((C, D_FEAT), jnp.float32),
        pltpu.SemaphoreType.DMA,
        pltpu.SemaphoreType.DMA,
        pltpu.SemaphoreType.DMA,
    ]


@functools.partial(
    pl.kernel,
    out_type=jax.ShapeDtypeStruct((N_EDGES, D_FEAT), jnp.float32),
    mesh=_mesh,
    scratch_types=_scratch,
)
def _edge_grad(x_hbm, negx_hbm, src_hbm, dst_hbm, out_hbm, idx_s, idx_d, *scr):
    rows = [scr[4 * b + 0] for b in range(NBUF)]
    sem_g = [scr[4 * b + 1] for b in range(NBUF)]
    sem_a = [scr[4 * b + 2] for b in range(NBUF)]
    sem_o = [scr[4 * b + 3] for b in range(NBUF)]

    wid = lax.axis_index("s") * 2 + lax.axis_index("c")
    base0 = wid * E_PER_W
    pltpu.sync_copy(src_hbm.at[wid], idx_s)
    pltpu.sync_copy(dst_hbm.at[wid], idx_d)

    def group_body(g, carry):
        ks = [g * NBUF + b for b in range(NBUF)]
        d_g, d_a, d_o = [], [], []
        for b in range(NBUF):
            d_g.append(
                pltpu.async_copy(x_hbm.at[idx_d.at[ks[b]]], rows[b], sem_g[b]))
        for b in range(NBUF):
            d_g[b].wait()
            d_a.append(
                pltpu.async_copy(negx_hbm.at[idx_s.at[ks[b]]], rows[b], sem_a[b],
                                 add=True))
        for b in range(NBUF):
            d_a[b].wait()
            d_o.append(
                pltpu.async_copy(rows[b], out_hbm.at[pl.ds(base0 + ks[b] * C, C)],
                                 sem_o[b]))
        for b in range(NBUF):
            d_o[b].wait()
        return carry

    lax.fori_loop(0, NGROUP, group_body, 0, unroll=False)


def kernel(x, edge_index):
    negx = -x
    src = edge_index[0].reshape(NW, NCHUNK_W, C)
    dst = edge_index[1].reshape(NW, NCHUNK_W, C)
    return _edge_grad(x, negx, src, dst)


# speedup vs baseline: 1.0370x; 1.0370x over previous
import functools

import jax
import jax.numpy as jnp
from jax import lax
from jax.experimental import pallas as pl
from jax.experimental.pallas import tpu as pltpu
from jax.experimental.pallas import tpu_sc as plsc

N_NODES = 10000
N_EDGES = 320000
D_FEAT = 128

NW = 32
C = 128
NCHUNKS = N_EDGES // C
BASE_TRIPS = NCHUNKS // NW
EXTRA = NCHUNKS % NW
NBUF = 6
NSLOT = 84

_mesh = plsc.VectorSubcoreMesh(core_axis_name="c", subcore_axis_name="s")


def _neg_body(x_ref, o_ref):
    o_ref[...] = -x_ref[...]


def _negate(x):
    return pl.pallas_call(
        _neg_body,
        out_shape=jax.ShapeDtypeStruct((N_NODES, D_FEAT), jnp.float32),
    )(x)


_scratch = []
for _b in range(NBUF):
    _scratch += [
        pltpu.VMEM((C,), jnp.int32),
        pltpu.VMEM((C,), jnp.int32),
        pltpu.VMEM((C, D_FEAT), jnp.float32),
        pltpu.SemaphoreType.DMA,
        pltpu.SemaphoreType.DMA,
        pltpu.SemaphoreType.DMA,
        pltpu.SemaphoreType.DMA,
    ]


@functools.partial(
    pl.kernel,
    out_type=jax.ShapeDtypeStruct((N_EDGES, D_FEAT), jnp.float32),
    mesh=_mesh,
    scratch_types=_scratch,
)
def _edge_grad(x_hbm, negx_hbm, src_hbm, dst_hbm, out_hbm, *scr):
    idx_s = [scr[7 * b + 0] for b in range(NBUF)]
    idx_d = [scr[7 * b + 1] for b in range(NBUF)]
    rows = [scr[7 * b + 2] for b in range(NBUF)]
    sem_i = [scr[7 * b + 3] for b in range(NBUF)]
    sem_g = [scr[7 * b + 4] for b in range(NBUF)]
    sem_a = [scr[7 * b + 5] for b in range(NBUF)]
    sem_o = [scr[7 * b + 6] for b in range(NBUF)]

    wid = lax.axis_index("s") * 2 + lax.axis_index("c")
    chunk0 = BASE_TRIPS * wid + lax.min(wid, EXTRA)
    trips = BASE_TRIPS + jnp.where(wid < EXTRA, 1, 0)

    def group_body(g, carry):
        js = [g * NBUF + b for b in range(NBUF)]
        live = [js[b] < trips for b in range(NBUF)]
        bases = [(chunk0 + js[b]) * C for b in range(NBUF)]
        for b in range(NBUF):
            @pl.when(live[b])
            def _(b=b):
                pltpu.async_copy(src_hbm.at[pl.ds(bases[b], C)], idx_s[b], sem_i[b])
                pltpu.async_copy(dst_hbm.at[pl.ds(bases[b], C)], idx_d[b], sem_i[b])
        for b in range(NBUF):
            @pl.when(live[b])
            def _(b=b):
                pltpu.make_async_copy(
                    src_hbm.at[pl.ds(bases[b], C)], idx_s[b], sem_i[b]).wait()
                pltpu.make_async_copy(
                    dst_hbm.at[pl.ds(bases[b], C)], idx_d[b], sem_i[b]).wait()
                pltpu.async_copy(x_hbm.at[idx_d[b]], rows[b], sem_g[b])
        for b in range(NBUF):
            @pl.when(live[b])
            def _(b=b):
                pltpu.make_async_copy(x_hbm.at[idx_d[b]], rows[b], sem_g[b]).wait()
                pltpu.make_async_copy(
                    negx_hbm.at[idx_s[b]], rows[b], sem_a[b]).start(add=True)
        for b in range(NBUF):
            @pl.when(live[b])
            def _(b=b):
                pltpu.make_async_copy(negx_hbm.at[idx_s[b]], rows[b], sem_a[b]).wait()
                pltpu.async_copy(rows[b], out_hbm.at[pl.ds(bases[b], C)], sem_o[b])
        for b in range(NBUF):
            @pl.when(live[b])
            def _(b=b):
                pltpu.make_async_copy(
                    rows[b], out_hbm.at[pl.ds(bases[b], C)], sem_o[b]).wait()
        return carry

    lax.fori_loop(0, NSLOT // NBUF, group_body, 0, unroll=False)


def kernel(x, edge_index):
    negx = _negate(x)
    src = edge_index[0]
    dst = edge_index[1]
    return _edge_grad(x, negx, src, dst)
